# baseline (speedup 1.0000x reference)
import jax
import jax.numpy as jnp
from jax import lax
from jax.experimental import pallas as pl
from jax.experimental.pallas import tpu as pltpu
from jax.experimental.pallas import tpu_sc as plsc

A = 10001
E = 320000
NB = 32
H = 128
B = 200
L = 50

NW = 32
A_PAD = 10240
APW = A_PAD // NW
AG = 4
NGA = APW // AG
GB = 64
NGB = 160
E_PAD = NW * NGB * GB
BPW = E_PAD // NW

_f32 = jnp.float32

def _sc_mesh():
    return plsc.VectorSubcoreMesh(core_axis_name="c", subcore_axis_name="s")


def _wid():
    return lax.axis_index("s") * 2 + lax.axis_index("c")


def _sc_agg(mb, a2b_r):
    def body(mb_hbm, idx_hbm, out_hbm, idx_v,
             b0, b1, b2, b3, ov0, ov1,
             s0, s1, s2, s3, sw0, sw1):
        w = _wid()
        pltpu.sync_copy(idx_hbm.at[w], idx_v)
        gb = (b0, b1, b2, b3)
        gs = (s0, s1, s2, s3)
        ovs = (ov0, ov1)
        sws = (sw0, sw1)

        def issue(g, p):
            pltpu.async_copy(mb_hbm.at[idx_v.at[g]], gb[p], gs[p])

        def wait_gather(g, p):
            pltpu.make_async_copy(mb_hbm.at[idx_v.at[g]], gb[p], gs[p]).wait()

        def compute(g, p, q, half):
            buf = gb[p]
            ov = ovs[q]
            for a in range(AG):
                r0 = a * NB

                def cstep(c, carry):
                    vs = [buf[r0 + k, pl.ds(c * 16, 16)] for k in range(NB)]
                    acc = [((vs[i] + vs[i + 8]) + vs[i + 16]) + vs[i + 24]
                           for i in range(8)]
                    b4 = [acc[i] + acc[i + 4] for i in range(4)]
                    bb2 = [b4[0] + b4[2], b4[1] + b4[3]]
                    ssum = bb2[0] + bb2[1]
                    m = vs[0]
                    for k in range(1, NB):
                        m = jnp.maximum(m, vs[k])
                    ov[half * AG + a, pl.ds(c * 16, 16)] = ssum * m
                    return carry

                lax.fori_loop(0, 8, cstep, 0)

        issue(0, 0)
        issue(1, 1)
        issue(2, 2)

        def step(i, carry):
            for j in range(4):
                g = 4 * i + j
                q = (j // 2) % 2
                wait_gather(g, j)

                @pl.when(g + 3 < NGA)
                def _():
                    issue(g + 3, (j + 3) % 4)

                if j % 2 == 0:
                    p_pair = 2 * i + j // 2

                    @pl.when(p_pair >= 2)
                    def _():
                        pltpu.make_async_copy(
                            ovs[q],
                            out_hbm.at[pl.ds(w * APW + (p_pair - 2) * 2 * AG,
                                             2 * AG)],
                            sws[q]).wait()

                compute(g, j, q, j % 2)
                if j % 2 == 1:
                    p_pair = 2 * i + j // 2
                    pltpu.async_copy(
                        ovs[q],
                        out_hbm.at[pl.ds(w * APW + p_pair * 2 * AG, 2 * AG)],
                        sws[q])
            return carry

        lax.fori_loop(0, NGA // 4, step, 0)
        for q, p_pair in ((0, NGA // 2 - 2), (1, NGA // 2 - 1)):
            pltpu.make_async_copy(
                ovs[q],
                out_hbm.at[pl.ds(w * APW + p_pair * 2 * AG, 2 * AG)],
                sws[q]).wait()

    f = pl.kernel(
        body,
        out_type=jax.ShapeDtypeStruct((A_PAD, H), _f32),
        mesh=_sc_mesh(),
        scratch_types=[
            pltpu.VMEM((NGA, 128), jnp.int32),
            pltpu.VMEM((128, H), _f32),
            pltpu.VMEM((128, H), _f32),
            pltpu.VMEM((128, H), _f32),
            pltpu.VMEM((128, H), _f32),
            pltpu.VMEM((2 * AG, H), _f32),
            pltpu.VMEM((2 * AG, H), _f32),
            pltpu.SemaphoreType.DMA,
            pltpu.SemaphoreType.DMA,
            pltpu.SemaphoreType.DMA,
            pltpu.SemaphoreType.DMA,
            pltpu.SemaphoreType.DMA,
            pltpu.SemaphoreType.DMA,
        ],
    )
    return f(mb, a2b_r)


def _sc_ydiff(aw, mb, b2a_r, b2revb_r):
    def body(aw_hbm, mb_hbm, ia_hbm, ir_hbm, y_hbm, ia_v, ir_v,
             a0, a1, a2, a3, r0_, r1_, r2_, r3_, by0, by1,
             sa0, sa1, sa2, sa3, sr0, sr1, sr2, sr3, sw0, sw1):
        w = _wid()
        pltpu.sync_copy(ia_hbm.at[w], ia_v)
        pltpu.sync_copy(ir_hbm.at[w], ir_v)
        ab = (a0, a1, a2, a3)
        rb = (r0_, r1_, r2_, r3_)
        sas = (sa0, sa1, sa2, sa3)
        srs = (sr0, sr1, sr2, sr3)
        bys = (by0, by1)
        sws = (sw0, sw1)

        def issue(g, p):
            pltpu.async_copy(aw_hbm.at[ia_v.at[g]], ab[p], sas[p])
            pltpu.async_copy(mb_hbm.at[ir_v.at[g]], rb[p], srs[p])

        def wait_gather(g, p):
            pltpu.make_async_copy(aw_hbm.at[ia_v.at[g]], ab[p], sas[p]).wait()
            pltpu.make_async_copy(mb_hbm.at[ir_v.at[g]], rb[p], srs[p]).wait()

        issue(0, 0)
        issue(1, 1)
        issue(2, 2)

        def step(i, carry):
            for j in range(4):
                g = 4 * i + j
                q = j % 2
                wait_gather(g, j)

                @pl.when(g + 3 < NGB)
                def _():
                    issue(g + 3, (j + 3) % 4)

                @pl.when(g >= 2)
                def _():
                    pltpu.make_async_copy(
                        bys[q], y_hbm.at[pl.ds(w * BPW + (g - 2) * GB, GB)],
                        sws[q]).wait()

                ba, br, by = ab[j], rb[j], bys[q]

                def rstep(r, c2):
                    for c in range(8):
                        by[r, pl.ds(c * 16, 16)] = (
                            ba[r, pl.ds(c * 16, 16)] - br[r, pl.ds(c * 16, 16)])
                    return c2

                lax.fori_loop(0, GB, rstep, 0)
                pltpu.async_copy(
                    by, y_hbm.at[pl.ds(w * BPW + g * GB, GB)], sws[q])
            return carry

        lax.fori_loop(0, NGB // 4, step, 0)
        for g in (NGB - 2, NGB - 1):
            pltpu.make_async_copy(
                bys[g % 2], y_hbm.at[pl.ds(w * BPW + g * GB, GB)],
                sws[g % 2]).wait()

    f = pl.kernel(
        body,
        out_type=jax.ShapeDtypeStruct((E_PAD, H), _f32),
        mesh=_sc_mesh(),
        scratch_types=[
            pltpu.VMEM((NGB, GB), jnp.int32),
            pltpu.VMEM((NGB, GB), jnp.int32),
            pltpu.VMEM((GB, H), _f32),
            pltpu.VMEM((GB, H), _f32),
            pltpu.VMEM((GB, H), _f32),
            pltpu.VMEM((GB, H), _f32),
            pltpu.VMEM((GB, H), _f32),
            pltpu.VMEM((GB, H), _f32),
            pltpu.VMEM((GB, H), _f32),
            pltpu.VMEM((GB, H), _f32),
            pltpu.VMEM((GB, H), _f32),
            pltpu.VMEM((GB, H), _f32),
            pltpu.SemaphoreType.DMA,
            pltpu.SemaphoreType.DMA,
            pltpu.SemaphoreType.DMA,
            pltpu.SemaphoreType.DMA,
            pltpu.SemaphoreType.DMA,
            pltpu.SemaphoreType.DMA,
            pltpu.SemaphoreType.DMA,
            pltpu.SemaphoreType.DMA,
            pltpu.SemaphoreType.DMA,
            pltpu.SemaphoreType.DMA,
        ],
    )
    return f(aw, mb, b2a_r, b2revb_r)


def _mm_relu(x, wt, bk):
    M, K = x.shape
    N = wt.shape[1]

    def body(x_ref, w_ref, o_ref):
        o_ref[...] = jnp.maximum(
            jnp.dot(x_ref[...], w_ref[...], preferred_element_type=_f32), 0.0)

    return pl.pallas_call(
        body,
        grid=(M // bk,),
        in_specs=[pl.BlockSpec((bk, K), lambda i: (i, 0)),
                  pl.BlockSpec((K, N), lambda i: (0, 0))],
        out_specs=pl.BlockSpec((bk, N), lambda i: (i, 0)),
        out_shape=jax.ShapeDtypeStruct((M, N), _f32),
    )(x, wt)


def _atom_update(ma, agg):
    bk = 1280

    def body(ma_ref, agg_ref, mo_ref):
        mo_ref[...] = ma_ref[...] + agg_ref[...]

    return pl.pallas_call(
        body,
        grid=(A_PAD // bk,),
        in_specs=[pl.BlockSpec((bk, H), lambda i: (i, 0)),
                  pl.BlockSpec((bk, H), lambda i: (i, 0))],
        out_specs=pl.BlockSpec((bk, H), lambda i: (i, 0)),
        out_shape=jax.ShapeDtypeStruct((A_PAD, H), _f32),
    )(ma, agg)


def _bond_update(ib, y, wt):
    bk = 2560

    def body(ib_ref, y_ref, w_ref, o_ref):
        o_ref[...] = jnp.maximum(
            ib_ref[...] + jnp.dot(y_ref[...], w_ref[...],
                                  preferred_element_type=_f32), 0.0)

    return pl.pallas_call(
        body,
        grid=(E_PAD // bk,),
        in_specs=[pl.BlockSpec((bk, H), lambda i: (i, 0)),
                  pl.BlockSpec((bk, H), lambda i: (i, 0)),
                  pl.BlockSpec((H, H), lambda i: (0, 0))],
        out_specs=pl.BlockSpec((bk, H), lambda i: (i, 0)),
        out_shape=jax.ShapeDtypeStruct((E_PAD, H), _f32),
    )(ib, y, wt)


def _lr_combine(agg2, ma, ia, w0, w1, w2):
    bk = 1280

    def body(a_ref, m_ref, i_ref, w0_ref, w1_ref, w2_ref, hid_ref, msg_ref):
        h = (jnp.dot(a_ref[...], w0_ref[...], preferred_element_type=_f32)
             + jnp.dot(m_ref[...], w1_ref[...], preferred_element_type=_f32)
             + jnp.dot(i_ref[...], w2_ref[...], preferred_element_type=_f32))
        hid_ref[...] = h
        msg_ref[...] = jnp.maximum(h, 0.0)

    return pl.pallas_call(
        body,
        grid=(A_PAD // bk,),
        in_specs=[pl.BlockSpec((bk, H), lambda i: (i, 0)),
                  pl.BlockSpec((bk, H), lambda i: (i, 0)),
                  pl.BlockSpec((bk, H), lambda i: (i, 0)),
                  pl.BlockSpec((H, H), lambda i: (0, 0)),
                  pl.BlockSpec((H, H), lambda i: (0, 0)),
                  pl.BlockSpec((H, H), lambda i: (0, 0))],
        out_specs=[pl.BlockSpec((bk, H), lambda i: (i, 0)),
                   pl.BlockSpec((bk, H), lambda i: (i, 0))],
        out_shape=[jax.ShapeDtypeStruct((A_PAD, H), _f32),
                   jax.ShapeDtypeStruct((A_PAD, H), _f32)],
    )(agg2, ma, ia, w0, w1, w2)


def _gru(x_t, hid_t, wih_f, whh_f, bih_f, bhh_f, wih_r, whh_r, bih_r, bhh_r,
         wf, wr, wob):
    def body(x_ref, hid_ref, wihf_ref, whhf_ref, bihf_ref, bhhf_ref,
             wihr_ref, whhr_ref, bihr_ref, bhhr_ref, wf_ref, wr_ref, wob_ref,
             o_ref, fbuf):
        def cell(h, xt, wih, whh, bih, bhh):
            gi = jnp.dot(xt, wih, preferred_element_type=_f32) + bih
            gh = jnp.dot(h, whh, preferred_element_type=_f32) + bhh
            r = jax.nn.sigmoid(gi[:, :H] + gh[:, :H])
            z = jax.nn.sigmoid(gi[:, H:2 * H] + gh[:, H:2 * H])
            n = jnp.tanh(gi[:, 2 * H:] + r * gh[:, 2 * H:])
            return (1.0 - z) * n + z * h

        h0 = lax.fori_loop(1, L, lambda t, acc: jnp.maximum(acc, hid_ref[t]),
                           hid_ref[0])

        def fstep(t, h):
            h = cell(h, x_ref[t], wihf_ref[...], whhf_ref[...],
                     bihf_ref[...], bhhf_ref[...])
            fbuf[t] = h
            return h

        lax.fori_loop(0, L, fstep, h0)

        def bstep(s, carry):
            hb, acc = carry
            t = L - 1 - s
            hb = cell(hb, x_ref[t], wihr_ref[...], whhr_ref[...],
                      bihr_ref[...], bhhr_ref[...])
            y = (jnp.dot(fbuf[t], wf_ref[...], preferred_element_type=_f32)
                 + jnp.dot(hb, wr_ref[...], preferred_element_type=_f32)
                 + wob_ref[...])
            return hb, acc + jnp.maximum(y, 0.0)

        _, acc = lax.fori_loop(0, L, bstep, (h0, jnp.zeros((B, H), _f32)))
        o_ref[...] = acc * (1.0 / L)

    return pl.pallas_call(
        body,
        out_shape=jax.ShapeDtypeStruct((B, H), _f32),
        scratch_shapes=[pltpu.VMEM((L, B, H), _f32)],
    )(x_t, hid_t, wih_f, whh_f, bih_f, bhh_f, wih_r, whh_r, bih_r, bhh_r,
      wf, wr, wob)


def kernel(f_atoms, f_bonds, a2b, b2a, b2revb, a_scope,
           W_i_atom, W_i_bond, W_h_0, W_h_1, lr_w, W_o_w, W_o_b,
           gru_wih_f, gru_whh_f, gru_bih_f, gru_bhh_f,
           gru_wih_r, gru_whh_r, gru_bih_r, gru_bhh_r):
    fa = jnp.pad(f_atoms, ((0, A_PAD - A), (0, 0)))
    fb = jnp.pad(f_bonds, ((0, E_PAD - E), (0, 0)))
    a2b_r = jnp.pad(a2b, ((0, A_PAD - A), (0, 0))).reshape(NW, NGA, 128)
    b2a_r = jnp.pad(b2a, (0, E_PAD - E)).reshape(NW, NGB, GB)
    b2revb_r = jnp.pad(b2revb, (0, E_PAD - E)).reshape(NW, NGB, GB)

    ia = _mm_relu(fa, W_i_atom.T, bk=1280)
    ib = _mm_relu(fb, W_i_bond.T, bk=2560)
    ma, mb = ia, ib
    for W_h in (W_h_0, W_h_1):
        agg = _sc_agg(mb, a2b_r)
        ma = _atom_update(ma, agg)
        y = _sc_ydiff(ma, mb, b2a_r, b2revb_r)
        mb = _bond_update(ib, y, W_h.T)
    agg2 = _sc_agg(mb, a2b_r)
    hid, msg = _lr_combine(agg2, ma, ia,
                           lr_w[:, :H].T, lr_w[:, H:2 * H].T, lr_w[:, 2 * H:].T)
    x_t = msg[1:1 + B * L].reshape(B, L, H).transpose(1, 0, 2)
    hid_t = hid[1:1 + B * L].reshape(B, L, H).transpose(1, 0, 2)
    return _gru(x_t, hid_t,
                gru_wih_f.T, gru_whh_f.T,
                gru_bih_f.reshape(1, -1), gru_bhh_f.reshape(1, -1),
                gru_wih_r.T, gru_whh_r.T,
                gru_bih_r.reshape(1, -1), gru_bhh_r.reshape(1, -1),
                W_o_w[:, :H].T, W_o_w[:, H:].T, W_o_b.reshape(1, -1))


# speedup vs baseline: 1.0917x; 1.0917x over previous
import jax
import jax.numpy as jnp
from jax import lax
from jax.experimental import pallas as pl
from jax.experimental.pallas import tpu as pltpu
from jax.experimental.pallas import tpu_sc as plsc

A = 10001
E = 320000
NB = 32
H = 128
B = 200
L = 50

NW = 32
A_PAD = 10240
APW = A_PAD // NW
AG = 8
NGA = APW // AG
GB = 128
NGB = 80
E_PAD = NW * NGB * GB
BPW = E_PAD // NW

_f32 = jnp.float32

def _sc_mesh():
    return plsc.VectorSubcoreMesh(core_axis_name="c", subcore_axis_name="s")


def _wid():
    return lax.axis_index("s") * 2 + lax.axis_index("c")


def _sc_agg(mb, a2b_r):
    def body(mb_hbm, idx_hbm, out_hbm, idx_v,
             ba0, bb0, ba1, bb1, ov0, ov1,
             sa0, sb0, sa1, sb1, sw0, sw1):
        w = _wid()
        pltpu.sync_copy(idx_hbm.at[w], idx_v)
        bufs = ((ba0, bb0, sa0, sb0), (ba1, bb1, sa1, sb1))
        ovs = (ov0, ov1)
        sws = (sw0, sw1)

        def issue(g, p):
            ba, bb, sa, sb = bufs[p]
            pltpu.async_copy(mb_hbm.at[idx_v.at[2 * g]], ba, sa)
            pltpu.async_copy(mb_hbm.at[idx_v.at[2 * g + 1]], bb, sb)

        def consume(g, p):
            ba, bb, sa, sb = bufs[p]
            ov = ovs[p]
            pltpu.make_async_copy(mb_hbm.at[idx_v.at[2 * g]], ba, sa).wait()
            pltpu.make_async_copy(mb_hbm.at[idx_v.at[2 * g + 1]], bb, sb).wait()

            @pl.when(g >= 2)
            def _():
                pltpu.make_async_copy(
                    ov, out_hbm.at[pl.ds(w * APW + (g - 2) * AG, AG)],
                    sws[p]).wait()

            for a in range(AG):
                buf = ba if a < 4 else bb
                r0 = (a % 4) * NB

                def cstep(c, carry):
                    vs = [buf[r0 + k, pl.ds(c * 16, 16)] for k in range(NB)]
                    acc = [((vs[i] + vs[i + 8]) + vs[i + 16]) + vs[i + 24]
                           for i in range(8)]
                    b4 = [acc[i] + acc[i + 4] for i in range(4)]
                    b2 = [b4[0] + b4[2], b4[1] + b4[3]]
                    ssum = b2[0] + b2[1]
                    m = vs[0]
                    for k in range(1, NB):
                        m = jnp.maximum(m, vs[k])
                    ov[a, pl.ds(c * 16, 16)] = ssum * m
                    return carry

                lax.fori_loop(0, 8, cstep, 0)
            pltpu.async_copy(
                ov, out_hbm.at[pl.ds(w * APW + g * AG, AG)], sws[p])

        issue(0, 0)

        def pairstep(i, carry):
            g0 = 2 * i
            issue(g0 + 1, 1)
            consume(g0, 0)

            @pl.when(g0 + 2 < NGA)
            def _():
                issue(g0 + 2, 0)

            consume(g0 + 1, 1)
            return carry

        lax.fori_loop(0, NGA // 2, pairstep, 0)
        for g in (NGA - 2, NGA - 1):
            pltpu.make_async_copy(
                ovs[g % 2], out_hbm.at[pl.ds(w * APW + g * AG, AG)],
                sws[g % 2]).wait()

    f = pl.kernel(
        body,
        out_type=jax.ShapeDtypeStruct((A_PAD, H), _f32),
        mesh=_sc_mesh(),
        scratch_types=[
            pltpu.VMEM((2 * NGA, 128), jnp.int32),
            pltpu.VMEM((128, H), _f32),
            pltpu.VMEM((128, H), _f32),
            pltpu.VMEM((128, H), _f32),
            pltpu.VMEM((128, H), _f32),
            pltpu.VMEM((AG, H), _f32),
            pltpu.VMEM((AG, H), _f32),
            pltpu.SemaphoreType.DMA,
            pltpu.SemaphoreType.DMA,
            pltpu.SemaphoreType.DMA,
            pltpu.SemaphoreType.DMA,
            pltpu.SemaphoreType.DMA,
            pltpu.SemaphoreType.DMA,
        ],
    )
    return f(mb, a2b_r)


def _sc_ydiff(aw, mb, b2a_r, b2revb_r):
    def body(aw_hbm, mb_hbm, ia_hbm, ir_hbm, y_hbm, ia_v, ir_v,
             a0, a1, r0_, r1_, by0, by1,
             sa0, sa1, sr0, sr1, sw0, sw1):
        w = _wid()
        pltpu.sync_copy(ia_hbm.at[w], ia_v)
        pltpu.sync_copy(ir_hbm.at[w], ir_v)
        ab = (a0, a1)
        rb = (r0_, r1_)
        sas = (sa0, sa1)
        srs = (sr0, sr1)
        bys = (by0, by1)
        sws = (sw0, sw1)

        def issue(g, p):
            pltpu.async_copy(aw_hbm.at[ia_v.at[g]], ab[p], sas[p])
            pltpu.async_copy(mb_hbm.at[ir_v.at[g]], rb[p], srs[p])

        def consume(g, p):
            ba, br, by = ab[p], rb[p], bys[p]
            pltpu.make_async_copy(aw_hbm.at[ia_v.at[g]], ba, sas[p]).wait()
            pltpu.make_async_copy(mb_hbm.at[ir_v.at[g]], br, srs[p]).wait()

            @pl.when(g >= 2)
            def _():
                pltpu.make_async_copy(
                    by, y_hbm.at[pl.ds(w * BPW + (g - 2) * GB, GB)],
                    sws[p]).wait()

            def rstep(r, c2):
                for c in range(8):
                    by[r, pl.ds(c * 16, 16)] = (
                        ba[r, pl.ds(c * 16, 16)] - br[r, pl.ds(c * 16, 16)])
                return c2

            lax.fori_loop(0, GB, rstep, 0)
            pltpu.async_copy(
                by, y_hbm.at[pl.ds(w * BPW + g * GB, GB)], sws[p])

        issue(0, 0)

        def pairstep(i, carry):
            g0 = 2 * i
            issue(g0 + 1, 1)
            consume(g0, 0)

            @pl.when(g0 + 2 < NGB)
            def _():
                issue(g0 + 2, 0)

            consume(g0 + 1, 1)
            return carry

        lax.fori_loop(0, NGB // 2, pairstep, 0)
        for g in (NGB - 2, NGB - 1):
            pltpu.make_async_copy(
                bys[g % 2], y_hbm.at[pl.ds(w * BPW + g * GB, GB)],
                sws[g % 2]).wait()

    f = pl.kernel(
        body,
        out_type=jax.ShapeDtypeStruct((E_PAD, H), _f32),
        mesh=_sc_mesh(),
        scratch_types=[
            pltpu.VMEM((NGB, GB), jnp.int32),
            pltpu.VMEM((NGB, GB), jnp.int32),
            pltpu.VMEM((GB, H), _f32),
            pltpu.VMEM((GB, H), _f32),
            pltpu.VMEM((GB, H), _f32),
            pltpu.VMEM((GB, H), _f32),
            pltpu.VMEM((GB, H), _f32),
            pltpu.VMEM((GB, H), _f32),
            pltpu.SemaphoreType.DMA,
            pltpu.SemaphoreType.DMA,
            pltpu.SemaphoreType.DMA,
            pltpu.SemaphoreType.DMA,
            pltpu.SemaphoreType.DMA,
            pltpu.SemaphoreType.DMA,
        ],
    )
    return f(aw, mb, b2a_r, b2revb_r)


def _mm_relu(x, wt, bk):
    M, K = x.shape
    N = wt.shape[1]

    def body(x_ref, w_ref, o_ref):
        o_ref[...] = jnp.maximum(
            jnp.dot(x_ref[...], w_ref[...], preferred_element_type=_f32), 0.0)

    return pl.pallas_call(
        body,
        grid=(M // bk,),
        in_specs=[pl.BlockSpec((bk, K), lambda i: (i, 0)),
                  pl.BlockSpec((K, N), lambda i: (0, 0))],
        out_specs=pl.BlockSpec((bk, N), lambda i: (i, 0)),
        out_shape=jax.ShapeDtypeStruct((M, N), _f32),
    )(x, wt)


def _atom_update(ma, agg):
    bk = 1280

    def body(ma_ref, agg_ref, mo_ref):
        mo_ref[...] = ma_ref[...] + agg_ref[...]

    return pl.pallas_call(
        body,
        grid=(A_PAD // bk,),
        in_specs=[pl.BlockSpec((bk, H), lambda i: (i, 0)),
                  pl.BlockSpec((bk, H), lambda i: (i, 0))],
        out_specs=pl.BlockSpec((bk, H), lambda i: (i, 0)),
        out_shape=jax.ShapeDtypeStruct((A_PAD, H), _f32),
    )(ma, agg)


def _bond_update(ib, y, wt):
    bk = 2560

    def body(ib_ref, y_ref, w_ref, o_ref):
        o_ref[...] = jnp.maximum(
            ib_ref[...] + jnp.dot(y_ref[...], w_ref[...],
                                  preferred_element_type=_f32), 0.0)

    return pl.pallas_call(
        body,
        grid=(E_PAD // bk,),
        in_specs=[pl.BlockSpec((bk, H), lambda i: (i, 0)),
                  pl.BlockSpec((bk, H), lambda i: (i, 0)),
                  pl.BlockSpec((H, H), lambda i: (0, 0))],
        out_specs=pl.BlockSpec((bk, H), lambda i: (i, 0)),
        out_shape=jax.ShapeDtypeStruct((E_PAD, H), _f32),
    )(ib, y, wt)


def _lr_combine(agg2, ma, ia, w0, w1, w2):
    bk = 1280

    def body(a_ref, m_ref, i_ref, w0_ref, w1_ref, w2_ref, hid_ref, msg_ref):
        h = (jnp.dot(a_ref[...], w0_ref[...], preferred_element_type=_f32)
             + jnp.dot(m_ref[...], w1_ref[...], preferred_element_type=_f32)
             + jnp.dot(i_ref[...], w2_ref[...], preferred_element_type=_f32))
        hid_ref[...] = h
        msg_ref[...] = jnp.maximum(h, 0.0)

    return pl.pallas_call(
        body,
        grid=(A_PAD // bk,),
        in_specs=[pl.BlockSpec((bk, H), lambda i: (i, 0)),
                  pl.BlockSpec((bk, H), lambda i: (i, 0)),
                  pl.BlockSpec((bk, H), lambda i: (i, 0)),
                  pl.BlockSpec((H, H), lambda i: (0, 0)),
                  pl.BlockSpec((H, H), lambda i: (0, 0)),
                  pl.BlockSpec((H, H), lambda i: (0, 0))],
        out_specs=[pl.BlockSpec((bk, H), lambda i: (i, 0)),
                   pl.BlockSpec((bk, H), lambda i: (i, 0))],
        out_shape=[jax.ShapeDtypeStruct((A_PAD, H), _f32),
                   jax.ShapeDtypeStruct((A_PAD, H), _f32)],
    )(agg2, ma, ia, w0, w1, w2)


def _gru(x_t, hid_t, wih_f, whh_f, bih_f, bhh_f, wih_r, whh_r, bih_r, bhh_r,
         wf, wr, wob):
    def body(x_ref, hid_ref, wihf_ref, whhf_ref, bihf_ref, bhhf_ref,
             wihr_ref, whhr_ref, bihr_ref, bhhr_ref, wf_ref, wr_ref, wob_ref,
             o_ref, fbuf):
        def cell(h, xt, wih, whh, bih, bhh):
            gi = jnp.dot(xt, wih, preferred_element_type=_f32) + bih
            gh = jnp.dot(h, whh, preferred_element_type=_f32) + bhh
            r = jax.nn.sigmoid(gi[:, :H] + gh[:, :H])
            z = jax.nn.sigmoid(gi[:, H:2 * H] + gh[:, H:2 * H])
            n = jnp.tanh(gi[:, 2 * H:] + r * gh[:, 2 * H:])
            return (1.0 - z) * n + z * h

        h0 = lax.fori_loop(1, L, lambda t, acc: jnp.maximum(acc, hid_ref[t]),
                           hid_ref[0])

        def fstep(t, h):
            h = cell(h, x_ref[t], wihf_ref[...], whhf_ref[...],
                     bihf_ref[...], bhhf_ref[...])
            fbuf[t] = h
            return h

        lax.fori_loop(0, L, fstep, h0)

        def bstep(s, carry):
            hb, acc = carry
            t = L - 1 - s
            hb = cell(hb, x_ref[t], wihr_ref[...], whhr_ref[...],
                      bihr_ref[...], bhhr_ref[...])
            y = (jnp.dot(fbuf[t], wf_ref[...], preferred_element_type=_f32)
                 + jnp.dot(hb, wr_ref[...], preferred_element_type=_f32)
                 + wob_ref[...])
            return hb, acc + jnp.maximum(y, 0.0)

        _, acc = lax.fori_loop(0, L, bstep, (h0, jnp.zeros((B, H), _f32)))
        o_ref[...] = acc * (1.0 / L)

    return pl.pallas_call(
        body,
        out_shape=jax.ShapeDtypeStruct((B, H), _f32),
        scratch_shapes=[pltpu.VMEM((L, B, H), _f32)],
    )(x_t, hid_t, wih_f, whh_f, bih_f, bhh_f, wih_r, whh_r, bih_r, bhh_r,
      wf, wr, wob)


def kernel(f_atoms, f_bonds, a2b, b2a, b2revb, a_scope,
           W_i_atom, W_i_bond, W_h_0, W_h_1, lr_w, W_o_w, W_o_b,
           gru_wih_f, gru_whh_f, gru_bih_f, gru_bhh_f,
           gru_wih_r, gru_whh_r, gru_bih_r, gru_bhh_r):
    fa = jnp.pad(f_atoms, ((0, A_PAD - A), (0, 0)))
    fb = jnp.pad(f_bonds, ((0, E_PAD - E), (0, 0)))
    a2b_r = jnp.pad(a2b, ((0, A_PAD - A), (0, 0))).reshape(NW, 2 * NGA, 128)
    b2a_r = jnp.pad(b2a, (0, E_PAD - E)).reshape(NW, NGB, GB)
    b2revb_r = jnp.pad(b2revb, (0, E_PAD - E)).reshape(NW, NGB, GB)

    ia = _mm_relu(fa, W_i_atom.T, bk=1280)
    ib = _mm_relu(fb, W_i_bond.T, bk=2560)
    ma, mb = ia, ib
    for W_h in (W_h_0, W_h_1):
        agg = _sc_agg(mb, a2b_r)
        ma = _atom_update(ma, agg)
        y = _sc_ydiff(ma, mb, b2a_r, b2revb_r)
        mb = _bond_update(ib, y, W_h.T)
    agg2 = _sc_agg(mb, a2b_r)
    hid, msg = _lr_combine(agg2, ma, ia,
                           lr_w[:, :H].T, lr_w[:, H:2 * H].T, lr_w[:, 2 * H:].T)
    x_t = msg[1:1 + B * L].reshape(B, L, H).transpose(1, 0, 2)
    hid_t = hid[1:1 + B * L].reshape(B, L, H).transpose(1, 0, 2)
    return _gru(x_t, hid_t,
                gru_wih_f.T, gru_whh_f.T,
                gru_bih_f.reshape(1, -1), gru_bhh_f.reshape(1, -1),
                gru_wih_r.T, gru_whh_r.T,
                gru_bih_r.reshape(1, -1), gru_bhh_r.reshape(1, -1),
                W_o_w[:, :H].T, W_o_w[:, H:].T, W_o_b.reshape(1, -1))


# speedup vs baseline: 1.0940x; 1.0022x over previous
import jax
import jax.numpy as jnp
from jax import lax
from jax.experimental import pallas as pl
from jax.experimental.pallas import tpu as pltpu
from jax.experimental.pallas import tpu_sc as plsc

A = 10001
E = 320000
NB = 32
H = 128
B = 200
L = 50

NW = 32
A_PAD = 10240
APW = A_PAD // NW
AG = 8
NGA = APW // AG
GB = 128
NGB = 80
E_PAD = NW * NGB * GB
BPW = E_PAD // NW

_f32 = jnp.float32

def _sc_mesh():
    return plsc.VectorSubcoreMesh(core_axis_name="c", subcore_axis_name="s")


def _wid():
    return lax.axis_index("s") * 2 + lax.axis_index("c")


def _sc_agg(mb, a2b_r):
    def body(mb_hbm, idx_hbm, out_hbm, idx_v,
             ba0, bb0, ba1, bb1, ba2, bb2, ov0, ov1, ov2,
             sa0, sb0, sa1, sb1, sa2, sb2, sw0, sw1, sw2):
        w = _wid()
        pltpu.sync_copy(idx_hbm.at[w], idx_v)
        bufs = ((ba0, bb0, sa0, sb0), (ba1, bb1, sa1, sb1),
                (ba2, bb2, sa2, sb2))
        ovs = (ov0, ov1, ov2)
        sws = (sw0, sw1, sw2)

        def issue(g, p):
            ba, bb, sa, sb = bufs[p]
            pltpu.async_copy(mb_hbm.at[idx_v.at[2 * g]], ba, sa)
            pltpu.async_copy(mb_hbm.at[idx_v.at[2 * g + 1]], bb, sb)

        def consume(g, p):
            ba, bb, sa, sb = bufs[p]
            ov = ovs[p]
            pltpu.make_async_copy(mb_hbm.at[idx_v.at[2 * g]], ba, sa).wait()
            pltpu.make_async_copy(mb_hbm.at[idx_v.at[2 * g + 1]], bb, sb).wait()

            @pl.when(g >= 3)
            def _():
                pltpu.make_async_copy(
                    ov, out_hbm.at[pl.ds(w * APW + (g - 3) * AG, AG)],
                    sws[p]).wait()

            for a in range(AG):
                buf = ba if a < 4 else bb
                r0 = (a % 4) * NB

                def cstep(c, carry):
                    vs = [buf[r0 + k, pl.ds(c * 16, 16)] for k in range(NB)]
                    acc = [((vs[i] + vs[i + 8]) + vs[i + 16]) + vs[i + 24]
                           for i in range(8)]
                    b4 = [acc[i] + acc[i + 4] for i in range(4)]
                    b2 = [b4[0] + b4[2], b4[1] + b4[3]]
                    ssum = b2[0] + b2[1]
                    m = vs[0]
                    for k in range(1, NB):
                        m = jnp.maximum(m, vs[k])
                    ov[a, pl.ds(c * 16, 16)] = ssum * m
                    return carry

                lax.fori_loop(0, 8, cstep, 0)
            pltpu.async_copy(
                ov, out_hbm.at[pl.ds(w * APW + g * AG, AG)], sws[p])

        issue(0, 0)
        issue(1, 1)

        def tristep(i, carry):
            for j in range(3):
                g = 3 * i + j

                @pl.when(g + 2 < NGA)
                def _():
                    issue(g + 2, (j + 2) % 3)

                consume(g, j)
            return carry

        lax.fori_loop(0, (NGA - 1) // 3, tristep, 0)
        consume(NGA - 1, (NGA - 1) % 3)
        for g in (NGA - 3, NGA - 2, NGA - 1):
            pltpu.make_async_copy(
                ovs[g % 3], out_hbm.at[pl.ds(w * APW + g * AG, AG)],
                sws[g % 3]).wait()

    f = pl.kernel(
        body,
        out_type=jax.ShapeDtypeStruct((A_PAD, H), _f32),
        mesh=_sc_mesh(),
        scratch_types=[
            pltpu.VMEM((2 * NGA, 128), jnp.int32),
            pltpu.VMEM((128, H), _f32),
            pltpu.VMEM((128, H), _f32),
            pltpu.VMEM((128, H), _f32),
            pltpu.VMEM((128, H), _f32),
            pltpu.VMEM((128, H), _f32),
            pltpu.VMEM((128, H), _f32),
            pltpu.VMEM((AG, H), _f32),
            pltpu.VMEM((AG, H), _f32),
            pltpu.VMEM((AG, H), _f32),
            pltpu.SemaphoreType.DMA,
            pltpu.SemaphoreType.DMA,
            pltpu.SemaphoreType.DMA,
            pltpu.SemaphoreType.DMA,
            pltpu.SemaphoreType.DMA,
            pltpu.SemaphoreType.DMA,
            pltpu.SemaphoreType.DMA,
            pltpu.SemaphoreType.DMA,
            pltpu.SemaphoreType.DMA,
        ],
    )
    return f(mb, a2b_r)


def _sc_ydiff(aw, mb, b2a_r, b2revb_r):
    def body(aw_hbm, mb_hbm, ia_hbm, ir_hbm, y_hbm, ia_v, ir_v,
             a0, a1, r0_, r1_, by0, by1,
             sa0, sa1, sr0, sr1, sw0, sw1):
        w = _wid()
        pltpu.sync_copy(ia_hbm.at[w], ia_v)
        pltpu.sync_copy(ir_hbm.at[w], ir_v)
        ab = (a0, a1)
        rb = (r0_, r1_)
        sas = (sa0, sa1)
        srs = (sr0, sr1)
        bys = (by0, by1)
        sws = (sw0, sw1)

        def issue(g, p):
            pltpu.async_copy(aw_hbm.at[ia_v.at[g]], ab[p], sas[p])
            pltpu.async_copy(mb_hbm.at[ir_v.at[g]], rb[p], srs[p])

        def consume(g, p):
            ba, br, by = ab[p], rb[p], bys[p]
            pltpu.make_async_copy(aw_hbm.at[ia_v.at[g]], ba, sas[p]).wait()
            pltpu.make_async_copy(mb_hbm.at[ir_v.at[g]], br, srs[p]).wait()

            @pl.when(g >= 2)
            def _():
                pltpu.make_async_copy(
                    by, y_hbm.at[pl.ds(w * BPW + (g - 2) * GB, GB)],
                    sws[p]).wait()

            def rstep(r, c2):
                for c in range(8):
                    by[r, pl.ds(c * 16, 16)] = (
                        ba[r, pl.ds(c * 16, 16)] - br[r, pl.ds(c * 16, 16)])
                return c2

            lax.fori_loop(0, GB, rstep, 0)
            pltpu.async_copy(
                by, y_hbm.at[pl.ds(w * BPW + g * GB, GB)], sws[p])

        issue(0, 0)

        def pairstep(i, carry):
            g0 = 2 * i
            issue(g0 + 1, 1)
            consume(g0, 0)

            @pl.when(g0 + 2 < NGB)
            def _():
                issue(g0 + 2, 0)

            consume(g0 + 1, 1)
            return carry

        lax.fori_loop(0, NGB // 2, pairstep, 0)
        for g in (NGB - 2, NGB - 1):
            pltpu.make_async_copy(
                bys[g % 2], y_hbm.at[pl.ds(w * BPW + g * GB, GB)],
                sws[g % 2]).wait()

    f = pl.kernel(
        body,
        out_type=jax.ShapeDtypeStruct((E_PAD, H), _f32),
        mesh=_sc_mesh(),
        scratch_types=[
            pltpu.VMEM((NGB, GB), jnp.int32),
            pltpu.VMEM((NGB, GB), jnp.int32),
            pltpu.VMEM((GB, H), _f32),
            pltpu.VMEM((GB, H), _f32),
            pltpu.VMEM((GB, H), _f32),
            pltpu.VMEM((GB, H), _f32),
            pltpu.VMEM((GB, H), _f32),
            pltpu.VMEM((GB, H), _f32),
            pltpu.SemaphoreType.DMA,
            pltpu.SemaphoreType.DMA,
            pltpu.SemaphoreType.DMA,
            pltpu.SemaphoreType.DMA,
            pltpu.SemaphoreType.DMA,
            pltpu.SemaphoreType.DMA,
        ],
    )
    return f(aw, mb, b2a_r, b2revb_r)


def _mm_relu(x, wt, bk):
    M, K = x.shape
    N = wt.shape[1]

    def body(x_ref, w_ref, o_ref):
        o_ref[...] = jnp.maximum(
            jnp.dot(x_ref[...], w_ref[...], preferred_element_type=_f32), 0.0)

    return pl.pallas_call(
        body,
        grid=(M // bk,),
        in_specs=[pl.BlockSpec((bk, K), lambda i: (i, 0)),
                  pl.BlockSpec((K, N), lambda i: (0, 0))],
        out_specs=pl.BlockSpec((bk, N), lambda i: (i, 0)),
        out_shape=jax.ShapeDtypeStruct((M, N), _f32),
    )(x, wt)


def _atom_update(ma, agg):
    bk = 1280

    def body(ma_ref, agg_ref, mo_ref):
        mo_ref[...] = ma_ref[...] + agg_ref[...]

    return pl.pallas_call(
        body,
        grid=(A_PAD // bk,),
        in_specs=[pl.BlockSpec((bk, H), lambda i: (i, 0)),
                  pl.BlockSpec((bk, H), lambda i: (i, 0))],
        out_specs=pl.BlockSpec((bk, H), lambda i: (i, 0)),
        out_shape=jax.ShapeDtypeStruct((A_PAD, H), _f32),
    )(ma, agg)


def _bond_update(ib, y, wt):
    bk = 2560

    def body(ib_ref, y_ref, w_ref, o_ref):
        o_ref[...] = jnp.maximum(
            ib_ref[...] + jnp.dot(y_ref[...], w_ref[...],
                                  preferred_element_type=_f32), 0.0)

    return pl.pallas_call(
        body,
        grid=(E_PAD // bk,),
        in_specs=[pl.BlockSpec((bk, H), lambda i: (i, 0)),
                  pl.BlockSpec((bk, H), lambda i: (i, 0)),
                  pl.BlockSpec((H, H), lambda i: (0, 0))],
        out_specs=pl.BlockSpec((bk, H), lambda i: (i, 0)),
        out_shape=jax.ShapeDtypeStruct((E_PAD, H), _f32),
    )(ib, y, wt)


def _lr_combine(agg2, ma, ia, w0, w1, w2):
    bk = 1280

    def body(a_ref, m_ref, i_ref, w0_ref, w1_ref, w2_ref, hid_ref, msg_ref):
        h = (jnp.dot(a_ref[...], w0_ref[...], preferred_element_type=_f32)
             + jnp.dot(m_ref[...], w1_ref[...], preferred_element_type=_f32)
             + jnp.dot(i_ref[...], w2_ref[...], preferred_element_type=_f32))
        hid_ref[...] = h
        msg_ref[...] = jnp.maximum(h, 0.0)

    return pl.pallas_call(
        body,
        grid=(A_PAD // bk,),
        in_specs=[pl.BlockSpec((bk, H), lambda i: (i, 0)),
                  pl.BlockSpec((bk, H), lambda i: (i, 0)),
                  pl.BlockSpec((bk, H), lambda i: (i, 0)),
                  pl.BlockSpec((H, H), lambda i: (0, 0)),
                  pl.BlockSpec((H, H), lambda i: (0, 0)),
                  pl.BlockSpec((H, H), lambda i: (0, 0))],
        out_specs=[pl.BlockSpec((bk, H), lambda i: (i, 0)),
                   pl.BlockSpec((bk, H), lambda i: (i, 0))],
        out_shape=[jax.ShapeDtypeStruct((A_PAD, H), _f32),
                   jax.ShapeDtypeStruct((A_PAD, H), _f32)],
    )(agg2, ma, ia, w0, w1, w2)


def _gru(x_t, hid_t, wih_f, whh_f, bih_f, bhh_f, wih_r, whh_r, bih_r, bhh_r,
         wf, wr, wob):
    def body(x_ref, hid_ref, wihf_ref, whhf_ref, bihf_ref, bhhf_ref,
             wihr_ref, whhr_ref, bihr_ref, bhhr_ref, wf_ref, wr_ref, wob_ref,
             o_ref, fbuf):
        def cell(h, xt, wih, whh, bih, bhh):
            gi = jnp.dot(xt, wih, preferred_element_type=_f32) + bih
            gh = jnp.dot(h, whh, preferred_element_type=_f32) + bhh
            r = jax.nn.sigmoid(gi[:, :H] + gh[:, :H])
            z = jax.nn.sigmoid(gi[:, H:2 * H] + gh[:, H:2 * H])
            n = jnp.tanh(gi[:, 2 * H:] + r * gh[:, 2 * H:])
            return (1.0 - z) * n + z * h

        h0 = lax.fori_loop(1, L, lambda t, acc: jnp.maximum(acc, hid_ref[t]),
                           hid_ref[0])

        def fstep(t, h):
            h = cell(h, x_ref[t], wihf_ref[...], whhf_ref[...],
                     bihf_ref[...], bhhf_ref[...])
            fbuf[t] = h
            return h

        lax.fori_loop(0, L, fstep, h0)

        def bstep(s, carry):
            hb, acc = carry
            t = L - 1 - s
            hb = cell(hb, x_ref[t], wihr_ref[...], whhr_ref[...],
                      bihr_ref[...], bhhr_ref[...])
            y = (jnp.dot(fbuf[t], wf_ref[...], preferred_element_type=_f32)
                 + jnp.dot(hb, wr_ref[...], preferred_element_type=_f32)
                 + wob_ref[...])
            return hb, acc + jnp.maximum(y, 0.0)

        _, acc = lax.fori_loop(0, L, bstep, (h0, jnp.zeros((B, H), _f32)))
        o_ref[...] = acc * (1.0 / L)

    return pl.pallas_call(
        body,
        out_shape=jax.ShapeDtypeStruct((B, H), _f32),
        scratch_shapes=[pltpu.VMEM((L, B, H), _f32)],
    )(x_t, hid_t, wih_f, whh_f, bih_f, bhh_f, wih_r, whh_r, bih_r, bhh_r,
      wf, wr, wob)


def kernel(f_atoms, f_bonds, a2b, b2a, b2revb, a_scope,
           W_i_atom, W_i_bond, W_h_0, W_h_1, lr_w, W_o_w, W_o_b,
           gru_wih_f, gru_whh_f, gru_bih_f, gru_bhh_f,
           gru_wih_r, gru_whh_r, gru_bih_r, gru_bhh_r):
    fa = jnp.pad(f_atoms, ((0, A_PAD - A), (0, 0)))
    fb = jnp.pad(f_bonds, ((0, E_PAD - E), (0, 0)))
    a2b_r = jnp.pad(a2b, ((0, A_PAD - A), (0, 0))).reshape(NW, 2 * NGA, 128)
    b2a_r = jnp.pad(b2a, (0, E_PAD - E)).reshape(NW, NGB, GB)
    b2revb_r = jnp.pad(b2revb, (0, E_PAD - E)).reshape(NW, NGB, GB)

    ia = _mm_relu(fa, W_i_atom.T, bk=1280)
    ib = _mm_relu(fb, W_i_bond.T, bk=2560)
    ma, mb = ia, ib
    for W_h in (W_h_0, W_h_1):
        agg = _sc_agg(mb, a2b_r)
        ma = _atom_update(ma, agg)
        y = _sc_ydiff(ma, mb, b2a_r, b2revb_r)
        mb = _bond_update(ib, y, W_h.T)
    agg2 = _sc_agg(mb, a2b_r)
    hid, msg = _lr_combine(agg2, ma, ia,
                           lr_w[:, :H].T, lr_w[:, H:2 * H].T, lr_w[:, 2 * H:].T)
    x_t = msg[1:1 + B * L].reshape(B, L, H).transpose(1, 0, 2)
    hid_t = hid[1:1 + B * L].reshape(B, L, H).transpose(1, 0, 2)
    return _gru(x_t, hid_t,
                gru_wih_f.T, gru_whh_f.T,
                gru_bih_f.reshape(1, -1), gru_bhh_f.reshape(1, -1),
                gru_wih_r.T, gru_whh_r.T,
                gru_bih_r.reshape(1, -1), gru_bhh_r.reshape(1, -1),
                W_o_w[:, :H].T, W_o_w[:, H:].T, W_o_b.reshape(1, -1))
